# baseline (speedup 1.0000x reference)
import jax
import jax.numpy as jnp
from jax.experimental import pallas as pl

_N = 21
_F = 128


def _fused_fwd(state_ref, adj_ref, W1_ref, b1_ref, W2_ref, b2_ref,
               Wf1_ref, bf1_ref, Wf2_ref, bf2_ref, Wf3_ref, bf3_ref,
               Wf4_ref, bf4_ref, Wpi_ref, bpi_ref, Wv_ref, bv_ref,
               pi_ref, v_ref):
    x0 = state_ref[0]
    a = adj_ref[0]
    a = a + jnp.eye(_N, dtype=a.dtype)
    deg = jnp.sum(a, axis=1, keepdims=True)
    s = jnp.where(deg > 0, jax.lax.rsqrt(deg), 0.0)

    z = s * jnp.dot(x0, W1_ref[...], preferred_element_type=jnp.float32)
    x = s * jnp.dot(a, z, preferred_element_type=jnp.float32) + b1_ref[...]
    z = s * jnp.dot(x, W2_ref[...], preferred_element_type=jnp.float32)
    x = s * jnp.dot(a, z, preferred_element_type=jnp.float32) + b2_ref[...]

    h = bf1_ref[...]
    xb = x.astype(jnp.bfloat16)
    for i in range(_N):
        h = h + jnp.dot(xb[i:i + 1, :], Wf1_ref[i * _N:(i + 1) * _N, :],
                        preferred_element_type=jnp.float32)
    h = jnp.maximum(h, 0.0)
    h = jnp.maximum(
        jnp.dot(h.astype(jnp.bfloat16), Wf2_ref[...],
                preferred_element_type=jnp.float32)
        + bf2_ref[...], 0.0)
    vx = jnp.maximum(
        jnp.dot(h, Wf3_ref[...], preferred_element_type=jnp.float32)
        + bf3_ref[...], 0.0)
    vx = jnp.maximum(
        jnp.dot(vx, Wf4_ref[...], preferred_element_type=jnp.float32)
        + bf4_ref[...], 0.0)

    pi_ref[...] = (jnp.dot(h, Wpi_ref[...], preferred_element_type=jnp.float32)
                   + bpi_ref[...])
    v_ref[...] = (jnp.dot(vx, Wv_ref[...], preferred_element_type=jnp.float32)
                  + bv_ref[...])


def kernel(state, adj, W1, b1, W2, b2, Wf1, bf1, Wf2, bf2, Wf3, bf3,
           Wf4, bf4, Wpi, bpi, Wv, bv):
    full = lambda x: pl.BlockSpec(x.shape, lambda i: tuple(0 for _ in x.shape))
    in_specs = [
        pl.BlockSpec((1, _N, _F), lambda i: (0, 0, 0)),
        pl.BlockSpec((1, _N, _N), lambda i: (0, 0, 0)),
    ]
    b1r, b2r = b1.reshape(1, _N), b2.reshape(1, _N)
    bf1r, bf2r = bf1.reshape(1, -1), bf2.reshape(1, -1)
    bf3r, bf4r = bf3.reshape(1, -1), bf4.reshape(1, -1)
    bpir, bvr = bpi.reshape(1, -1), bv.reshape(1, 1)
    rest = [W1, b1r, W2, b2r, Wf1.astype(jnp.bfloat16), bf1r,
            Wf2.astype(jnp.bfloat16), bf2r, Wf3, bf3r,
            Wf4, bf4r, Wpi, bpir, Wv, bvr]
    in_specs += [full(x) for x in rest]

    pi, v = pl.pallas_call(
        _fused_fwd,
        out_shape=(jax.ShapeDtypeStruct((1, 64), jnp.float32),
                   jax.ShapeDtypeStruct((1, 1), jnp.float32)),
        grid=(1,),
        in_specs=in_specs,
        out_specs=(pl.BlockSpec((1, 64), lambda i: (0, 0)),
                   pl.BlockSpec((1, 1), lambda i: (0, 0))),
    )(state, adj, *rest)
    return pi.reshape(64), v.reshape(1)


# speedup vs baseline: 1.1350x; 1.1350x over previous
import jax
import jax.numpy as jnp
from jax.experimental import pallas as pl
from jax.experimental.pallas import tpu as pltpu

_N = 21
_F = 128


def _fused_fwd(state_ref, adj_ref, W1_ref, b1_ref, W2_ref, b2_ref,
               Wf1_hbm, bf1_ref, Wf2_hbm, bf2_ref, Wf3_hbm, bf3_ref,
               Wf4_ref, bf4_ref, Wpi_hbm, bpi_ref, Wv_ref, bv_ref,
               pi_ref, v_ref,
               wf1_v, wf2_v, wf3_v, wpi_v, sems):
    c1 = pltpu.make_async_copy(Wf1_hbm.at[:, 0:512], wf1_v.at[:, 0:512],
                               sems.at[0])
    c2 = pltpu.make_async_copy(Wf1_hbm.at[:, 512:1024], wf1_v.at[:, 512:1024],
                               sems.at[1])
    c3 = pltpu.make_async_copy(Wf2_hbm.at[:, 0:256], wf2_v.at[:, 0:256],
                               sems.at[2])
    c4 = pltpu.make_async_copy(Wf2_hbm.at[:, 256:512], wf2_v.at[:, 256:512],
                               sems.at[3])
    c5 = pltpu.make_async_copy(Wf3_hbm, wf3_v, sems.at[4])
    c6 = pltpu.make_async_copy(Wpi_hbm, wpi_v, sems.at[5])
    c1.start()
    c2.start()
    c3.start()
    c4.start()
    c5.start()
    c6.start()

    x0 = state_ref[0]
    a = adj_ref[0]
    a = a + jnp.eye(_N, dtype=a.dtype)
    deg = jnp.sum(a, axis=1, keepdims=True)
    s = jnp.where(deg > 0, jax.lax.rsqrt(deg), 0.0)

    z = s * jnp.dot(x0, W1_ref[...], preferred_element_type=jnp.float32)
    x = s * jnp.dot(a, z, preferred_element_type=jnp.float32) + b1_ref[...]
    z = s * jnp.dot(x, W2_ref[...], preferred_element_type=jnp.float32)
    x = s * jnp.dot(a, z, preferred_element_type=jnp.float32) + b2_ref[...]

    c1.wait()
    c2.wait()
    parts = [jnp.dot(x[i:i + 1, :], wf1_v[i * _N:(i + 1) * _N, :],
                     preferred_element_type=jnp.float32) for i in range(_N)]
    parts.append(bf1_ref[...])
    while len(parts) > 1:
        nxt = [parts[i] + parts[i + 1] for i in range(0, len(parts) - 1, 2)]
        if len(parts) % 2:
            nxt.append(parts[-1])
        parts = nxt
    h = jnp.maximum(parts[0], 0.0)

    c3.wait()
    c4.wait()
    h = jnp.maximum(
        jnp.dot(h, wf2_v[...], preferred_element_type=jnp.float32)
        + bf2_ref[...], 0.0)
    c5.wait()
    c6.wait()
    vx = jnp.maximum(
        jnp.dot(h, wf3_v[...], preferred_element_type=jnp.float32)
        + bf3_ref[...], 0.0)
    vx = jnp.maximum(
        jnp.dot(vx, Wf4_ref[...], preferred_element_type=jnp.float32)
        + bf4_ref[...], 0.0)

    pi_ref[...] = (jnp.dot(h, wpi_v[...], preferred_element_type=jnp.float32)
                   + bpi_ref[...])
    v_ref[...] = (jnp.dot(vx, Wv_ref[...], preferred_element_type=jnp.float32)
                  + bv_ref[...])


def kernel(state, adj, W1, b1, W2, b2, Wf1, bf1, Wf2, bf2, Wf3, bf3,
           Wf4, bf4, Wpi, bpi, Wv, bv):
    full = lambda x: pl.BlockSpec(x.shape, lambda i: tuple(0 for _ in x.shape))
    hbm = pl.BlockSpec(memory_space=pltpu.MemorySpace.HBM)
    b1r, b2r = b1.reshape(1, _N), b2.reshape(1, _N)
    bf1r, bf2r = bf1.reshape(1, -1), bf2.reshape(1, -1)
    bf3r, bf4r = bf3.reshape(1, -1), bf4.reshape(1, -1)
    bpir, bvr = bpi.reshape(1, -1), bv.reshape(1, 1)
    rest = [W1, b1r, W2, b2r, Wf1, bf1r, Wf2, bf2r, Wf3, bf3r,
            Wf4, bf4r, Wpi, bpir, Wv, bvr]
    in_specs = [
        pl.BlockSpec((1, _N, _F), lambda i: (0, 0, 0)),
        pl.BlockSpec((1, _N, _N), lambda i: (0, 0, 0)),
        full(W1), full(b1r), full(W2), full(b2r),
        hbm, full(bf1r),
        hbm, full(bf2r),
        hbm, full(bf3r),
        full(Wf4), full(bf4r),
        hbm, full(bpir),
        full(Wv), full(bvr),
    ]

    pi, v = pl.pallas_call(
        _fused_fwd,
        out_shape=(jax.ShapeDtypeStruct((1, 64), jnp.float32),
                   jax.ShapeDtypeStruct((1, 1), jnp.float32)),
        grid=(1,),
        in_specs=in_specs,
        out_specs=(pl.BlockSpec((1, 64), lambda i: (0, 0)),
                   pl.BlockSpec((1, 1), lambda i: (0, 0))),
        scratch_shapes=[
            pltpu.VMEM((_N * _N, 1024), jnp.float32),
            pltpu.VMEM((1024, 512), jnp.float32),
            pltpu.VMEM((512, 256), jnp.float32),
            pltpu.VMEM((512, 64), jnp.float32),
            pltpu.SemaphoreType.DMA((6,)),
        ],
    )(state, adj, *rest)
    return pi.reshape(64), v.reshape(1)


# speedup vs baseline: 1.1619x; 1.0237x over previous
import jax
import jax.numpy as jnp
from jax.experimental import pallas as pl
from jax.experimental.pallas import tpu as pltpu

_N = 21
_F = 128


def _fused_fwd(state_hbm, adj_hbm, W1_hbm, b1_hbm, W2_hbm, b2_hbm,
               Wf1_hbm, bf1_hbm, Wf2_hbm, bf2_hbm, Wf3_hbm, bf3_hbm,
               Wf4_hbm, bf4_hbm, Wpi_hbm, bpi_hbm, Wv_hbm, bv_hbm,
               pi_ref, v_ref,
               x0_v, a_v, W1_v, b1_v, W2_v, b2_v, wf1_v, bf1_v, wf2_v, bf2_v,
               wf3_v, bf3_v, wf4_v, bf4_v, wpi_v, bpi_v, wv_v, bv_v, sems):
    srcs = [state_hbm.at[0], adj_hbm.at[0], W1_hbm, b1_hbm, W2_hbm, b2_hbm,
            Wf1_hbm, bf1_hbm, Wf2_hbm, bf2_hbm, Wf3_hbm, bf3_hbm,
            Wf4_hbm, bf4_hbm, Wpi_hbm, bpi_hbm, Wv_hbm, bv_hbm]
    dsts = [x0_v, a_v, W1_v, b1_v, W2_v, b2_v, wf1_v, bf1_v, wf2_v, bf2_v,
            wf3_v, bf3_v, wf4_v, bf4_v, wpi_v, bpi_v, wv_v, bv_v]
    cps = [pltpu.make_async_copy(s, d, sems.at[i])
           for i, (s, d) in enumerate(zip(srcs, dsts))]
    for c in cps:
        c.start()
    for c in cps[:6]:
        c.wait()

    x0 = x0_v[...]
    a = a_v[...] + jnp.eye(_N, dtype=jnp.float32)
    deg = jnp.sum(a, axis=1, keepdims=True)
    s = jnp.where(deg > 0, jax.lax.rsqrt(deg), 0.0)

    z = s * jnp.dot(x0, W1_v[...], preferred_element_type=jnp.float32)
    x = s * jnp.dot(a, z, preferred_element_type=jnp.float32) + b1_v[...]
    z = s * jnp.dot(x, W2_v[...], preferred_element_type=jnp.float32)
    x = s * jnp.dot(a, z, preferred_element_type=jnp.float32) + b2_v[...]

    cps[6].wait()
    cps[7].wait()
    parts = [jnp.dot(x[i:i + 1, :], wf1_v[i * _N:(i + 1) * _N, :],
                     preferred_element_type=jnp.float32) for i in range(_N)]
    parts.append(bf1_v[...])
    while len(parts) > 1:
        nxt = [parts[i] + parts[i + 1] for i in range(0, len(parts) - 1, 2)]
        if len(parts) % 2:
            nxt.append(parts[-1])
        parts = nxt
    h = jnp.maximum(parts[0], 0.0)

    cps[8].wait()
    cps[9].wait()
    h = jnp.maximum(
        jnp.dot(h, wf2_v[...], preferred_element_type=jnp.float32)
        + bf2_v[...], 0.0)
    cps[10].wait()
    cps[11].wait()
    vx = jnp.maximum(
        jnp.dot(h, wf3_v[...], preferred_element_type=jnp.float32)
        + bf3_v[...], 0.0)
    cps[12].wait()
    cps[13].wait()
    vx = jnp.maximum(
        jnp.dot(vx, wf4_v[...], preferred_element_type=jnp.float32)
        + bf4_v[...], 0.0)

    cps[14].wait()
    cps[15].wait()
    pi_ref[...] = (jnp.dot(h, wpi_v[...], preferred_element_type=jnp.float32)
                   + bpi_v[...])
    cps[16].wait()
    cps[17].wait()
    v_ref[...] = (jnp.dot(vx, wv_v[...], preferred_element_type=jnp.float32)
                  + bv_v[...])


def kernel(state, adj, W1, b1, W2, b2, Wf1, bf1, Wf2, bf2, Wf3, bf3,
           Wf4, bf4, Wpi, bpi, Wv, bv):
    hbm = pl.BlockSpec(memory_space=pltpu.MemorySpace.HBM)
    b1r, b2r = b1.reshape(1, _N), b2.reshape(1, _N)
    bf1r, bf2r = bf1.reshape(1, -1), bf2.reshape(1, -1)
    bf3r, bf4r = bf3.reshape(1, -1), bf4.reshape(1, -1)
    bpir, bvr = bpi.reshape(1, -1), bv.reshape(1, 1)
    args = [state, adj, W1, b1r, W2, b2r, Wf1, bf1r, Wf2, bf2r, Wf3, bf3r,
            Wf4, bf4r, Wpi, bpir, Wv, bvr]
    vmem = pltpu.VMEM
    f32 = jnp.float32
    scratch = [
        vmem((_N, _F), f32), vmem((_N, _N), f32),
        vmem((_F, _N), f32), vmem((1, _N), f32),
        vmem((_N, _N), f32), vmem((1, _N), f32),
        vmem((_N * _N, 1024), f32), vmem((1, 1024), f32),
        vmem((1024, 512), f32), vmem((1, 512), f32),
        vmem((512, 256), f32), vmem((1, 256), f32),
        vmem((256, 64), f32), vmem((1, 64), f32),
        vmem((512, 64), f32), vmem((1, 64), f32),
        vmem((64, 1), f32), vmem((1, 1), f32),
        pltpu.SemaphoreType.DMA((18,)),
    ]

    pi, v = pl.pallas_call(
        _fused_fwd,
        out_shape=(jax.ShapeDtypeStruct((1, 64), jnp.float32),
                   jax.ShapeDtypeStruct((1, 1), jnp.float32)),
        grid=(1,),
        in_specs=[hbm] * 18,
        out_specs=(pl.BlockSpec((1, 64), lambda i: (0, 0)),
                   pl.BlockSpec((1, 1), lambda i: (0, 0))),
        scratch_shapes=scratch,
    )(*args)
    return pi.reshape(64), v.reshape(1)


# speedup vs baseline: 1.4554x; 1.2526x over previous
import jax
import jax.numpy as jnp
from jax.experimental import pallas as pl

_N = 21
_F = 128

_ROWS = {}
_off = 0
for _name, _r in [('state0', 21), ('adj0', 21), ('W1', 128), ('W2', 21),
                  ('Wf4', 256), ('Wv', 64), ('b1', 1), ('b2', 1),
                  ('bf2', 4), ('bf3', 2), ('bf4', 1), ('bpi', 1),
                  ('bv', 1), ('bf1', 8)]:
    _ROWS[_name] = (_off, _off + _r)
    _off += _r
_PACK_ROWS = _off


def _fused_fwd(pack_ref, Wf1_ref, Wf2_ref, Wf3_ref, Wpi_ref, pi_ref, v_ref):
    def rows(name, cols=128):
        lo, hi = _ROWS[name]
        return pack_ref[lo:hi, 0:cols]

    def wide(name):
        lo, hi = _ROWS[name]
        return jnp.concatenate(
            [pack_ref[r:r + 1, :] for r in range(lo, hi)], axis=1)

    x0 = rows('state0')
    a = rows('adj0', _N) + jnp.eye(_N, dtype=jnp.float32)
    deg = jnp.sum(a, axis=1, keepdims=True)
    s = jnp.where(deg > 0, jax.lax.rsqrt(deg), 0.0)

    z = s * jnp.dot(x0, rows('W1', _N), preferred_element_type=jnp.float32)
    x = s * jnp.dot(a, z, preferred_element_type=jnp.float32) + rows('b1', _N)
    z = s * jnp.dot(x, rows('W2', _N), preferred_element_type=jnp.float32)
    x = s * jnp.dot(a, z, preferred_element_type=jnp.float32) + rows('b2', _N)

    parts = [jnp.dot(x[i:i + 1, :], Wf1_ref[i * _N:(i + 1) * _N, :],
                     preferred_element_type=jnp.float32) for i in range(_N)]
    parts.append(wide('bf1'))
    while len(parts) > 1:
        nxt = [parts[i] + parts[i + 1] for i in range(0, len(parts) - 1, 2)]
        if len(parts) % 2:
            nxt.append(parts[-1])
        parts = nxt
    h = jnp.maximum(parts[0], 0.0)

    h = jnp.maximum(
        jnp.dot(h, Wf2_ref[...], preferred_element_type=jnp.float32)
        + wide('bf2'), 0.0)
    vx = jnp.maximum(
        jnp.dot(h, Wf3_ref[...], preferred_element_type=jnp.float32)
        + wide('bf3'), 0.0)
    vx = jnp.maximum(
        jnp.dot(vx, rows('Wf4', 64), preferred_element_type=jnp.float32)
        + rows('bf4', 64), 0.0)

    pi_ref[...] = (jnp.dot(h, Wpi_ref[...], preferred_element_type=jnp.float32)
                   + rows('bpi', 64))
    v_ref[...] = (jnp.dot(vx, rows('Wv', 1), preferred_element_type=jnp.float32)
                  + rows('bv', 1))


def _pad128(m):
    return jnp.pad(m, ((0, 0), (0, 128 - m.shape[1])))


def kernel(state, adj, W1, b1, W2, b2, Wf1, bf1, Wf2, bf2, Wf3, bf3,
           Wf4, bf4, Wpi, bpi, Wv, bv):
    pack = jnp.concatenate([
        state[0],
        _pad128(adj[0]),
        _pad128(W1),
        _pad128(W2),
        _pad128(Wf4),
        _pad128(Wv),
        _pad128(b1[None, :]),
        _pad128(b2[None, :]),
        bf2.reshape(4, 128),
        bf3.reshape(2, 128),
        _pad128(bf4[None, :]),
        _pad128(bpi[None, :]),
        _pad128(bv[None, :]),
        bf1.reshape(8, 128),
    ], axis=0)

    full = lambda x: pl.BlockSpec(x.shape, lambda i: tuple(0 for _ in x.shape))
    pi, v = pl.pallas_call(
        _fused_fwd,
        out_shape=(jax.ShapeDtypeStruct((1, 64), jnp.float32),
                   jax.ShapeDtypeStruct((1, 1), jnp.float32)),
        grid=(1,),
        in_specs=[full(pack), full(Wf1), full(Wf2), full(Wf3), full(Wpi)],
        out_specs=(pl.BlockSpec((1, 64), lambda i: (0, 0)),
                   pl.BlockSpec((1, 1), lambda i: (0, 0))),
    )(pack, Wf1, Wf2, Wf3, Wpi)
    return pi.reshape(64), v.reshape(1)


# speedup vs baseline: 2.6621x; 1.8291x over previous
import jax
import jax.numpy as jnp
from jax.experimental import pallas as pl

_N = 21
_F = 128
_R_STATE, _R_ADJ, _R_W1, _R_W2, _R_END = 0, 21, 42, 170, 191


def _fused_fwd(pack_ref, Wf1_ref, Wf2_ref, Wf3_ref, Wpi_ref, Wf4_ref, Wv_ref,
               b1_ref, b2_ref, bf1_ref, bf2_ref, bf3_ref, bf4_ref,
               bpi_ref, bv_ref, pi_ref, v_ref):
    x0 = pack_ref[_R_STATE:_R_ADJ, :]
    a = (pack_ref[_R_ADJ:_R_W1, 0:_N]
         + jnp.eye(_N, dtype=jnp.float32))
    deg = jnp.sum(a, axis=1, keepdims=True)
    s = jnp.where(deg > 0, jax.lax.rsqrt(deg), 0.0)

    z = s * jnp.dot(x0, pack_ref[_R_W1:_R_W2, 0:_N],
                    preferred_element_type=jnp.float32)
    x = s * jnp.dot(a, z, preferred_element_type=jnp.float32) + b1_ref[...]
    z = s * jnp.dot(x, pack_ref[_R_W2:_R_END, 0:_N],
                    preferred_element_type=jnp.float32)
    x = s * jnp.dot(a, z, preferred_element_type=jnp.float32) + b2_ref[...]

    parts = [jnp.dot(x[i:i + 1, :], Wf1_ref[i * _N:(i + 1) * _N, :],
                     preferred_element_type=jnp.float32) for i in range(_N)]
    parts.append(bf1_ref[...])
    while len(parts) > 1:
        nxt = [parts[i] + parts[i + 1] for i in range(0, len(parts) - 1, 2)]
        if len(parts) % 2:
            nxt.append(parts[-1])
        parts = nxt
    h = jnp.maximum(parts[0], 0.0)

    h = jnp.maximum(
        jnp.dot(h, Wf2_ref[...], preferred_element_type=jnp.float32)
        + bf2_ref[...], 0.0)
    vx = jnp.maximum(
        jnp.dot(h, Wf3_ref[...], preferred_element_type=jnp.float32)
        + bf3_ref[...], 0.0)
    vx = jnp.maximum(
        jnp.dot(vx, Wf4_ref[...], preferred_element_type=jnp.float32)
        + bf4_ref[...], 0.0)

    pi_ref[...] = (jnp.dot(h, Wpi_ref[...], preferred_element_type=jnp.float32)
                   + bpi_ref[...])
    v_ref[...] = (jnp.dot(vx, Wv_ref[...], preferred_element_type=jnp.float32)
                  + bv_ref[...])


def _pad128(m):
    return jnp.pad(m, ((0, 0), (0, 128 - m.shape[1])))


def kernel(state, adj, W1, b1, W2, b2, Wf1, bf1, Wf2, bf2, Wf3, bf3,
           Wf4, bf4, Wpi, bpi, Wv, bv):
    pack = jnp.concatenate([
        state[0],
        _pad128(adj[0]),
        _pad128(W1),
        _pad128(W2),
    ], axis=0)
    args = [pack, Wf1, Wf2, Wf3, Wpi, Wf4, Wv,
            b1.reshape(1, -1), b2.reshape(1, -1), bf1.reshape(1, -1),
            bf2.reshape(1, -1), bf3.reshape(1, -1), bf4.reshape(1, -1),
            bpi.reshape(1, -1), bv.reshape(1, 1)]
    full = lambda x: pl.BlockSpec(x.shape, lambda i: tuple(0 for _ in x.shape))
    pi, v = pl.pallas_call(
        _fused_fwd,
        out_shape=(jax.ShapeDtypeStruct((1, 64), jnp.float32),
                   jax.ShapeDtypeStruct((1, 1), jnp.float32)),
        grid=(1,),
        in_specs=[full(x) for x in args],
        out_specs=(pl.BlockSpec((1, 64), lambda i: (0, 0)),
                   pl.BlockSpec((1, 1), lambda i: (0, 0))),
    )(*args)
    return pi.reshape(64), v.reshape(1)
